# baseline (speedup 1.0000x reference)
import functools

import jax
import jax.numpy as jnp
from jax import lax
from jax.experimental import pallas as pl
from jax.experimental.pallas import tpu as pltpu
from jax.experimental.pallas import tpu_sc as plsc

NC = 2
NS = 16
NW = NC * NS
LANES = 16
CW = 128

@functools.cache
def _sc_mesh():
  return plsc.VectorSubcoreMesh(
      core_axis_name="c", subcore_axis_name="s", num_cores=NC, num_subcores=NS)


def _make_deg_kernel(n_pad, cht):
  rows_per_tile = n_pad // NS

  @functools.partial(
      pl.kernel,
      out_type=jax.ShapeDtypeStruct((NC, n_pad), jnp.float32),
      mesh=_sc_mesh(),
      scratch_types=[
          pltpu.VMEM((cht, CW), jnp.int32),
          pltpu.VMEM((cht, CW), jnp.int32),
          pltpu.VMEM((CW,), jnp.float32),
          pltpu.VMEM((rows_per_tile,), jnp.float32),
          pltpu.VMEM_SHARED((n_pad,), jnp.float32),
      ],
  )
  def deg_kernel(src_hbm, dst_hbm, deg_out, sidx_v, didx_v, ones_v, zbuf_v,
                 hist_sh):
    c = lax.axis_index("c")
    s = lax.axis_index("s")
    wid = s * NC + c

    def zfill(i, carry):
      zbuf_v[pl.ds(i * LANES, LANES)] = jnp.zeros((LANES,), jnp.float32)
      return carry

    lax.fori_loop(0, rows_per_tile // LANES, zfill, 0)
    for t in range(CW // LANES):
      ones_v[pl.ds(t * LANES, LANES)] = jnp.ones((LANES,), jnp.float32)
    pltpu.sync_copy(zbuf_v, hist_sh.at[pl.ds(s * rows_per_tile,
                                             rows_per_tile)])
    plsc.subcore_barrier()

    base = wid * cht
    pltpu.sync_copy(src_hbm.at[pl.ds(base, cht)], sidx_v)
    pltpu.sync_copy(dst_hbm.at[pl.ds(base, cht)], didx_v)

    def body(j, carry):
      pltpu.sync_copy(ones_v, hist_sh.at[sidx_v.at[j]], add=True)
      pltpu.sync_copy(ones_v, hist_sh.at[didx_v.at[j]], add=True)
      return carry

    lax.fori_loop(0, cht, body, 0)
    plsc.subcore_barrier()
    pltpu.sync_copy(hist_sh.at[pl.ds(s * rows_per_tile, rows_per_tile)],
                    deg_out.at[c, pl.ds(s * rows_per_tile, rows_per_tile)])

  return deg_kernel


def _make_agg_kernel(n_pad, d, cht):
  rows_per_tile = n_pad // NS
  zrows = 32
  SB = 16
  nblk = cht // SB

  @functools.partial(
      pl.kernel,
      out_type=jax.ShapeDtypeStruct((NC, n_pad, d), jnp.float32),
      mesh=_sc_mesh(),
      scratch_types=[
          pltpu.VMEM((SB, CW), jnp.int32),
          pltpu.VMEM((SB, CW), jnp.int32),
          pltpu.VMEM((CW, d), jnp.float32),
          pltpu.VMEM((CW, d), jnp.float32),
          pltpu.VMEM((zrows, d), jnp.float32),
          pltpu.VMEM_SHARED((n_pad, d), jnp.float32),
          pltpu.SemaphoreType.DMA,
          pltpu.SemaphoreType.DMA,
          pltpu.SemaphoreType.DMA,
          pltpu.SemaphoreType.DMA,
      ],
  )
  def agg_kernel(zt_hbm, src_hbm, dst_hbm, acc_out, sidx_v, didx_v,
                 rows0_v, rows1_v, zrow_v, acc_sh, sem0, sem1, ssem0, ssem1):
    c = lax.axis_index("c")
    s = lax.axis_index("s")
    wid = s * NC + c

    def zfill(i, carry):
      for t in range(d // LANES):
        zrow_v[i, pl.ds(t * LANES, LANES)] = jnp.zeros((LANES,), jnp.float32)
      return carry

    lax.fori_loop(0, zrows, zfill, 0)

    def zcopy(k, carry):
      pltpu.sync_copy(
          zrow_v, acc_sh.at[pl.ds(s * rows_per_tile + k * zrows, zrows)])
      return carry

    lax.fori_loop(0, rows_per_tile // zrows, zcopy, 0)
    plsc.subcore_barrier()

    base = wid * cht

    def blk(b, carry):
      pltpu.sync_copy(src_hbm.at[pl.ds(base + b * SB, SB)], sidx_v)
      pltpu.sync_copy(dst_hbm.at[pl.ds(base + b * SB, SB)], didx_v)
      pltpu.async_copy(zt_hbm.at[didx_v.at[0]], rows0_v, sem0)
      pltpu.async_copy(zt_hbm.at[sidx_v.at[0]], rows1_v, sem1)

      def body(j, carry2):
        pltpu.make_async_copy(zt_hbm.at[didx_v.at[j]], rows0_v, sem0).wait()
        pltpu.async_copy(rows0_v, acc_sh.at[sidx_v.at[j]], ssem0, add=True)
        pltpu.make_async_copy(zt_hbm.at[sidx_v.at[j]], rows1_v, sem1).wait()
        pltpu.async_copy(rows1_v, acc_sh.at[didx_v.at[j]], ssem1, add=True)

        @pl.when(j + 1 < SB)
        def _():
          pltpu.make_async_copy(rows0_v, acc_sh.at[sidx_v.at[j]],
                                ssem0).wait()
          pltpu.async_copy(zt_hbm.at[didx_v.at[j + 1]], rows0_v, sem0)
          pltpu.make_async_copy(rows1_v, acc_sh.at[didx_v.at[j]],
                                ssem1).wait()
          pltpu.async_copy(zt_hbm.at[sidx_v.at[j + 1]], rows1_v, sem1)

        return carry2

      lax.fori_loop(0, SB, body, 0)
      pltpu.make_async_copy(rows0_v, acc_sh.at[sidx_v.at[0]], ssem0).wait()
      pltpu.make_async_copy(rows1_v, acc_sh.at[didx_v.at[0]], ssem1).wait()
      return carry

    lax.fori_loop(0, nblk, blk, 0)
    plsc.subcore_barrier()
    pltpu.sync_copy(acc_sh.at[pl.ds(s * rows_per_tile, rows_per_tile)],
                    acc_out.at[c, pl.ds(s * rows_per_tile, rows_per_tile)])

  return agg_kernel


def _head_body(deg0_ref, deg1_ref, x_ref, wi_ref, bi_ref, w0_ref, b0_ref,
               zt_ref, dis_ref):
  deg = deg0_ref[...] + deg1_ref[...]
  dis = jnp.where(deg > 0.0, lax.rsqrt(deg), 0.0)
  h = jnp.dot(x_ref[...], wi_ref[...],
              preferred_element_type=jnp.float32) + bi_ref[...]
  z = jnp.maximum(
      jnp.dot(h, w0_ref[...], preferred_element_type=jnp.float32)
      + b0_ref[...], 0.0)
  zt_ref[...] = z * dis
  dis_ref[...] = dis


def _mid_body(acc_ref0, acc_ref1, dis_ref, w1_ref, b1_ref, zt_ref):
  dis = dis_ref[...]
  a = (acc_ref0[0] + acc_ref1[0]) * dis
  z = jnp.maximum(
      jnp.dot(a, w1_ref[...], preferred_element_type=jnp.float32)
      + b1_ref[...], 0.0)
  zt_ref[...] = z * dis


def _out_body(acc_ref0, acc_ref1, dis_ref, wo_ref, bo_ref, o_ref):
  a = (acc_ref0[0] + acc_ref1[0]) * dis_ref[...]
  o_ref[...] = jnp.dot(a, wo_ref[...],
                       preferred_element_type=jnp.float32) + bo_ref[...]


def kernel(x, edge_index, W_in, b_in, W0, b0, W1, b1, W_out, b_out):
  n, d = x.shape
  e = edge_index.shape[1]
  n_pad = (n // 1024 + 1) * 1024
  cht = -(-e // (NW * CW))
  cht = -(-cht // 8) * 8
  e_pad = NW * CW * cht
  r = 2048
  grid = n_pad // r

  src = edge_index[0]
  dst = edge_index[1]
  npad_rows = n_pad - n
  pad_idx = n + (jnp.arange(e_pad - e, dtype=jnp.int32) % npad_rows)
  srcp = jnp.concatenate([src, pad_idx]).reshape(e_pad // CW, CW)
  dstp = jnp.concatenate([dst, pad_idx]).reshape(e_pad // CW, CW)
  x_pad = jnp.pad(x, ((0, npad_rows), (0, 0)))

  deg_parts = _make_deg_kernel(n_pad, cht)(srcp, dstp)
  deg0 = deg_parts[0].reshape(n_pad, 1)
  deg1 = deg_parts[1].reshape(n_pad, 1)

  col_spec = pl.BlockSpec((r, 1), lambda i: (i, 0))
  mat_spec = pl.BlockSpec((r, d), lambda i: (i, 0))
  w_spec = pl.BlockSpec((d, d), lambda i: (0, 0))
  b_spec = pl.BlockSpec((1, d), lambda i: (0, 0))
  acc_spec0 = pl.BlockSpec((1, r, d), lambda i: (0, i, 0))
  acc_spec1 = pl.BlockSpec((1, r, d), lambda i: (1, i, 0))

  zt0, dis = pl.pallas_call(
      _head_body,
      grid=(grid,),
      in_specs=[col_spec, col_spec, mat_spec, w_spec, b_spec, w_spec, b_spec],
      out_specs=[mat_spec, col_spec],
      out_shape=[
          jax.ShapeDtypeStruct((n_pad, d), jnp.float32),
          jax.ShapeDtypeStruct((n_pad, 1), jnp.float32),
      ],
  )(deg0, deg1, x_pad, W_in, b_in.reshape(1, d), W0, b0.reshape(1, d))

  agg = _make_agg_kernel(n_pad, d, cht)
  acc0 = agg(zt0, srcp, dstp)

  zt1 = pl.pallas_call(
      _mid_body,
      grid=(grid,),
      in_specs=[acc_spec0, acc_spec1, col_spec, w_spec, b_spec],
      out_specs=mat_spec,
      out_shape=jax.ShapeDtypeStruct((n_pad, d), jnp.float32),
  )(acc0, acc0, dis, W1, b1.reshape(1, d))

  acc1 = agg(zt1, srcp, dstp)

  out_pad = pl.pallas_call(
      _out_body,
      grid=(grid,),
      in_specs=[acc_spec0, acc_spec1, col_spec, w_spec, b_spec],
      out_specs=mat_spec,
      out_shape=jax.ShapeDtypeStruct((n_pad, d), jnp.float32),
  )(acc1, acc1, dis, W_out, b_out.reshape(1, d))

  return out_pad[:n]


# speedup vs baseline: 1.3484x; 1.3484x over previous
import functools

import jax
import jax.numpy as jnp
from jax import lax
from jax.experimental import pallas as pl
from jax.experimental.pallas import tpu as pltpu
from jax.experimental.pallas import tpu_sc as plsc

NC = 2
NS = 16
NW = NC * NS
LANES = 16
CW = 128

@functools.cache
def _sc_mesh():
  return plsc.VectorSubcoreMesh(
      core_axis_name="c", subcore_axis_name="s", num_cores=NC, num_subcores=NS)


def _make_deg_kernel(n_pad, cht):
  rows_per_tile = n_pad // NS

  @functools.partial(
      pl.kernel,
      out_type=jax.ShapeDtypeStruct((NC, n_pad), jnp.float32),
      mesh=_sc_mesh(),
      scratch_types=[
          pltpu.VMEM((cht, CW), jnp.int32),
          pltpu.VMEM((cht, CW), jnp.int32),
          pltpu.VMEM((CW,), jnp.float32),
          pltpu.VMEM((rows_per_tile,), jnp.float32),
          pltpu.VMEM_SHARED((n_pad,), jnp.float32),
      ],
  )
  def deg_kernel(src_hbm, dst_hbm, deg_out, sidx_v, didx_v, ones_v, zbuf_v,
                 hist_sh):
    c = lax.axis_index("c")
    s = lax.axis_index("s")
    wid = s * NC + c

    def zfill(i, carry):
      zbuf_v[pl.ds(i * LANES, LANES)] = jnp.zeros((LANES,), jnp.float32)
      return carry

    lax.fori_loop(0, rows_per_tile // LANES, zfill, 0)
    for t in range(CW // LANES):
      ones_v[pl.ds(t * LANES, LANES)] = jnp.ones((LANES,), jnp.float32)
    pltpu.sync_copy(zbuf_v, hist_sh.at[pl.ds(s * rows_per_tile,
                                             rows_per_tile)])
    plsc.subcore_barrier()

    base = wid * cht
    pltpu.sync_copy(src_hbm.at[pl.ds(base, cht)], sidx_v)
    pltpu.sync_copy(dst_hbm.at[pl.ds(base, cht)], didx_v)

    def body(j, carry):
      pltpu.sync_copy(ones_v, hist_sh.at[sidx_v.at[j]], add=True)
      pltpu.sync_copy(ones_v, hist_sh.at[didx_v.at[j]], add=True)
      return carry

    lax.fori_loop(0, cht, body, 0)
    plsc.subcore_barrier()
    pltpu.sync_copy(hist_sh.at[pl.ds(s * rows_per_tile, rows_per_tile)],
                    deg_out.at[c, pl.ds(s * rows_per_tile, rows_per_tile)])

  return deg_kernel


def _make_agg_kernel(n_pad, d, cht):
  rows_per_tile = n_pad // NS
  zrows = 32
  SB = 40
  nblk = cht // SB

  @functools.partial(
      pl.kernel,
      out_type=jax.ShapeDtypeStruct((NC, n_pad, d), jnp.float32),
      mesh=_sc_mesh(),
      scratch_types=[
          pltpu.VMEM((SB, CW), jnp.int32),
          pltpu.VMEM((SB, CW), jnp.int32),
          pltpu.VMEM((CW, d), jnp.float32),
          pltpu.VMEM((CW, d), jnp.float32),
          pltpu.VMEM((zrows, d), jnp.float32),
          pltpu.VMEM_SHARED((n_pad, d), jnp.float32),
          pltpu.SemaphoreType.DMA,
          pltpu.SemaphoreType.DMA,
      ],
  )
  def agg_kernel(zt_hbm, src_hbm, dst_hbm, acc_out, sidx_v, didx_v,
                 rows0_v, rows1_v, zrow_v, acc_sh, sem0, sem1):
    c = lax.axis_index("c")
    s = lax.axis_index("s")
    wid = s * NC + c

    def zfill(i, carry):
      for t in range(d // LANES):
        zrow_v[i, pl.ds(t * LANES, LANES)] = jnp.zeros((LANES,), jnp.float32)
      return carry

    lax.fori_loop(0, zrows, zfill, 0)

    def zcopy(k, carry):
      pltpu.sync_copy(
          zrow_v, acc_sh.at[pl.ds(s * rows_per_tile + k * zrows, zrows)])
      return carry

    lax.fori_loop(0, rows_per_tile // zrows, zcopy, 0)
    plsc.subcore_barrier()

    base = wid * cht

    def blk(b, carry):
      pltpu.sync_copy(src_hbm.at[pl.ds(base + b * SB, SB)], sidx_v)
      pltpu.sync_copy(dst_hbm.at[pl.ds(base + b * SB, SB)], didx_v)
      pltpu.async_copy(zt_hbm.at[didx_v.at[0]], rows0_v, sem0)
      pltpu.async_copy(zt_hbm.at[sidx_v.at[0]], rows1_v, sem1)

      def body(j, carry2):
        pltpu.make_async_copy(zt_hbm.at[didx_v.at[j]], rows0_v, sem0).wait()
        pltpu.sync_copy(rows0_v, acc_sh.at[sidx_v.at[j]], add=True)

        @pl.when(j + 1 < SB)
        def _():
          pltpu.async_copy(zt_hbm.at[didx_v.at[j + 1]], rows0_v, sem0)

        pltpu.make_async_copy(zt_hbm.at[sidx_v.at[j]], rows1_v, sem1).wait()
        pltpu.sync_copy(rows1_v, acc_sh.at[didx_v.at[j]], add=True)

        @pl.when(j + 1 < SB)
        def _():
          pltpu.async_copy(zt_hbm.at[sidx_v.at[j + 1]], rows1_v, sem1)

        return carry2

      lax.fori_loop(0, SB, body, 0)
      return carry

    lax.fori_loop(0, nblk, blk, 0)
    plsc.subcore_barrier()
    pltpu.sync_copy(acc_sh.at[pl.ds(s * rows_per_tile, rows_per_tile)],
                    acc_out.at[c, pl.ds(s * rows_per_tile, rows_per_tile)])

  return agg_kernel


def _head_body(deg0_ref, deg1_ref, x_ref, wi_ref, bi_ref, w0_ref, b0_ref,
               zt_ref, dis_ref):
  deg = deg0_ref[...] + deg1_ref[...]
  dis = jnp.where(deg > 0.0, lax.rsqrt(deg), 0.0)
  h = jnp.dot(x_ref[...], wi_ref[...],
              preferred_element_type=jnp.float32) + bi_ref[...]
  z = jnp.maximum(
      jnp.dot(h, w0_ref[...], preferred_element_type=jnp.float32)
      + b0_ref[...], 0.0)
  zt_ref[...] = z * dis
  dis_ref[...] = dis


def _mid_body(acc_ref0, acc_ref1, dis_ref, w1_ref, b1_ref, zt_ref):
  dis = dis_ref[...]
  a = (acc_ref0[0] + acc_ref1[0]) * dis
  z = jnp.maximum(
      jnp.dot(a, w1_ref[...], preferred_element_type=jnp.float32)
      + b1_ref[...], 0.0)
  zt_ref[...] = z * dis


def _out_body(acc_ref0, acc_ref1, dis_ref, wo_ref, bo_ref, o_ref):
  a = (acc_ref0[0] + acc_ref1[0]) * dis_ref[...]
  o_ref[...] = jnp.dot(a, wo_ref[...],
                       preferred_element_type=jnp.float32) + bo_ref[...]


def kernel(x, edge_index, W_in, b_in, W0, b0, W1, b1, W_out, b_out):
  n, d = x.shape
  e = edge_index.shape[1]
  n_pad = (n // 1024 + 1) * 1024
  cht = -(-e // (NW * CW))
  cht = -(-cht // 8) * 8
  e_pad = NW * CW * cht
  r = 2000
  grid = n // r

  src = edge_index[0]
  dst = edge_index[1]
  npad_rows = n_pad - n
  pad_idx = n + (jnp.arange(e_pad - e, dtype=jnp.int32) % npad_rows)
  srcp = jnp.concatenate([src, pad_idx]).reshape(e_pad // CW, CW)
  dstp = jnp.concatenate([dst, pad_idx]).reshape(e_pad // CW, CW)

  deg_parts = _make_deg_kernel(n_pad, cht)(srcp, dstp)
  deg0 = deg_parts[0].reshape(n_pad, 1)
  deg1 = deg_parts[1].reshape(n_pad, 1)

  col_spec = pl.BlockSpec((r, 1), lambda i: (i, 0))
  mat_spec = pl.BlockSpec((r, d), lambda i: (i, 0))
  w_spec = pl.BlockSpec((d, d), lambda i: (0, 0))
  b_spec = pl.BlockSpec((1, d), lambda i: (0, 0))
  acc_spec0 = pl.BlockSpec((1, r, d), lambda i: (0, i, 0))
  acc_spec1 = pl.BlockSpec((1, r, d), lambda i: (1, i, 0))

  zt0, dis = pl.pallas_call(
      _head_body,
      grid=(grid,),
      in_specs=[col_spec, col_spec, mat_spec, w_spec, b_spec, w_spec, b_spec],
      out_specs=[mat_spec, col_spec],
      out_shape=[
          jax.ShapeDtypeStruct((n_pad, d), jnp.float32),
          jax.ShapeDtypeStruct((n_pad, 1), jnp.float32),
      ],
  )(deg0, deg1, x, W_in, b_in.reshape(1, d), W0, b0.reshape(1, d))

  agg = _make_agg_kernel(n_pad, d, cht)
  acc0 = agg(zt0, srcp, dstp)

  zt1 = pl.pallas_call(
      _mid_body,
      grid=(grid,),
      in_specs=[acc_spec0, acc_spec1, col_spec, w_spec, b_spec],
      out_specs=mat_spec,
      out_shape=jax.ShapeDtypeStruct((n_pad, d), jnp.float32),
  )(acc0, acc0, dis, W1, b1.reshape(1, d))

  acc1 = agg(zt1, srcp, dstp)

  out = pl.pallas_call(
      _out_body,
      grid=(grid,),
      in_specs=[acc_spec0, acc_spec1, col_spec, w_spec, b_spec],
      out_specs=mat_spec,
      out_shape=jax.ShapeDtypeStruct((n, d), jnp.float32),
  )(acc1, acc1, dis, W_out, b_out.reshape(1, d))

  return out
